# baseline (speedup 1.0000x reference)
import functools

import jax
import jax.numpy as jnp
from jax.experimental import pallas as pl

LEVELS = 4
K = 1024
LATENT = 64
BLOCK_B = 512


def _rqvae_block(x_ref, We0, be0, We1, be1, We2, be2,
                 cb_ref, cbt_ref, cbn_ref,
                 Wd0, bd0, Wd1, bd1, Wd2, bd2,
                 dec_ref, r_ref, e_ref, cnt_ref, q_ref):
    f32 = jnp.float32
    bf16 = jnp.bfloat16

    def dot16(a, b):
        return jnp.dot(a.astype(bf16), b.astype(bf16), preferred_element_type=f32)

    x = x_ref[...]
    h = jnp.maximum(dot16(x, We0[...]) + be0[...], 0.0)
    h = jnp.maximum(dot16(h, We1[...]) + be1[...], 0.0)
    z = dot16(h, We2[...]) + be2[...]

    bs = z.shape[0]
    iota_k = jax.lax.broadcasted_iota(jnp.int32, (bs, K), 1)

    residual = z
    z_hat = jnp.zeros_like(z)
    idx_cols, cnt_rows = [], []
    for l in range(LEVELS):
        d = (-2.0) * dot16(residual, cbt_ref[l]) + cbn_ref[l]
        m = jnp.min(d, axis=1, keepdims=True)
        idx = jnp.min(jnp.where(d == m, iota_k, K), axis=1, keepdims=True)
        onehot = (iota_k == idx).astype(f32)
        e_l = jnp.dot(onehot, cb_ref[l], preferred_element_type=f32,
                      precision=jax.lax.Precision.HIGHEST)
        cnt_rows.append(jnp.sum(onehot, axis=0, keepdims=True).astype(jnp.int32))
        idx_cols.append(idx)
        r_ref[l] = residual
        e_ref[l] = e_l
        z_hat = z_hat + e_l
        residual = residual - e_l

    q_ref[...] = jnp.concatenate(idx_cols, axis=1)
    stacked = jnp.concatenate(cnt_rows, axis=0)

    @pl.when(pl.program_id(0) == 0)
    def _():
        cnt_ref[...] = stacked

    @pl.when(pl.program_id(0) != 0)
    def _():
        cnt_ref[...] = cnt_ref[...] + stacked

    zst = z + (z_hat - z)
    g = jnp.maximum(dot16(zst, Wd0[...]) + bd0[...], 0.0)
    g = jnp.maximum(dot16(g, Wd1[...]) + bd1[...], 0.0)
    dec_ref[...] = dot16(g, Wd2[...]) + bd2[...]


@functools.partial(jax.jit, static_argnames=())
def kernel(x, We0, be0, We1, be1, We2, be2, codebooks, Wd0, bd0, Wd1, bd1, Wd2, bd2):
    B, IN = x.shape
    bs = BLOCK_B
    grid = (B // bs,)
    cbt = codebooks.transpose(0, 2, 1)
    cbn = jnp.sum(codebooks * codebooks, axis=2)[:, None, :]
    H1 = We0.shape[1]
    H2 = We1.shape[1]
    OUT = Wd2.shape[1]

    def full(a):
        return pl.BlockSpec(a.shape, lambda i: (0,) * a.ndim)

    b2 = [b.reshape(1, -1) for b in (be0, be1, be2, bd0, bd1, bd2)]

    out_shapes = (
        jax.ShapeDtypeStruct((B, OUT), jnp.float32),
        jax.ShapeDtypeStruct((LEVELS, B, LATENT), jnp.float32),
        jax.ShapeDtypeStruct((LEVELS, B, LATENT), jnp.float32),
        jax.ShapeDtypeStruct((LEVELS, K), jnp.int32),
        jax.ShapeDtypeStruct((B, LEVELS), jnp.int32),
    )
    out_specs = (
        pl.BlockSpec((bs, OUT), lambda i: (i, 0)),
        pl.BlockSpec((LEVELS, bs, LATENT), lambda i: (0, i, 0)),
        pl.BlockSpec((LEVELS, bs, LATENT), lambda i: (0, i, 0)),
        pl.BlockSpec((LEVELS, K), lambda i: (0, 0)),
        pl.BlockSpec((bs, LEVELS), lambda i: (i, 0)),
    )
    in_specs = [
        pl.BlockSpec((bs, IN), lambda i: (i, 0)),
        full(We0), full(b2[0]), full(We1), full(b2[1]), full(We2), full(b2[2]),
        full(codebooks), full(cbt), full(cbn),
        full(Wd0), full(b2[3]), full(Wd1), full(b2[4]), full(Wd2), full(b2[5]),
    ]

    decoded, r, e, counts, quantized = pl.pallas_call(
        _rqvae_block,
        grid=grid,
        in_specs=in_specs,
        out_specs=out_specs,
        out_shape=out_shapes,
    )(x, We0, b2[0], We1, b2[1], We2, b2[2], codebooks, cbt, cbn,
      Wd0, b2[3], Wd1, b2[4], Wd2, b2[5])
    return (decoded, r, e, counts, quantized)


# speedup vs baseline: 1.4447x; 1.4447x over previous
import functools

import jax
import jax.numpy as jnp
from jax.experimental import pallas as pl

LEVELS = 4
K = 1024
LATENT = 64
BLOCK_B = 512


def _rqvae_block(x_ref, We0, be0, We1, be1, We2, be2,
                 cbh_ref, cbm_ref, cbl_ref, cbt_ref, cbn_ref,
                 Wd0, bd0, Wd1, bd1, Wd2, bd2,
                 dec_ref, r_ref, e_ref, cnt_ref, q_ref):
    f32 = jnp.float32
    bf16 = jnp.bfloat16

    def dot16(a, b):
        return jnp.dot(a.astype(bf16), b.astype(bf16), preferred_element_type=f32)

    x = x_ref[...]
    h = jnp.maximum(dot16(x, We0[...]) + be0[...], 0.0)
    h = jnp.maximum(dot16(h, We1[...]) + be1[...], 0.0)
    z = dot16(h, We2[...]) + be2[...]

    bs = z.shape[0]
    iota_k = jax.lax.broadcasted_iota(jnp.int32, (bs, K), 1)
    ones_row = jnp.ones((1, bs), dtype=bf16)

    residual = z
    z_hat = jnp.zeros_like(z)
    idx_cols, cnt_rows = [], []
    for l in range(LEVELS):
        d = dot16(residual, cbt_ref[l]) + cbn_ref[l]
        m = jnp.min(d, axis=1, keepdims=True)
        idx = jnp.min(jnp.where(d == m, iota_k, K), axis=1, keepdims=True)
        onehot = (iota_k == idx).astype(bf16)
        e_l = (jnp.dot(onehot, cbh_ref[l], preferred_element_type=f32)
               + jnp.dot(onehot, cbm_ref[l], preferred_element_type=f32)) \
              + jnp.dot(onehot, cbl_ref[l], preferred_element_type=f32)
        cnt_rows.append(jnp.dot(ones_row, onehot, preferred_element_type=f32))
        idx_cols.append(idx)
        r_ref[l] = residual
        e_ref[l] = e_l
        z_hat = z_hat + e_l
        residual = residual - e_l

    q_ref[...] = jnp.concatenate(idx_cols, axis=1)
    stacked = jnp.concatenate(cnt_rows, axis=0).astype(jnp.int32)

    @pl.when(pl.program_id(0) == 0)
    def _():
        cnt_ref[...] = stacked

    @pl.when(pl.program_id(0) != 0)
    def _():
        cnt_ref[...] = cnt_ref[...] + stacked

    zst = z + (z_hat - z)
    g = jnp.maximum(dot16(zst, Wd0[...]) + bd0[...], 0.0)
    g = jnp.maximum(dot16(g, Wd1[...]) + bd1[...], 0.0)
    dec_ref[...] = dot16(g, Wd2[...]) + bd2[...]


@functools.partial(jax.jit, static_argnames=())
def kernel(x, We0, be0, We1, be1, We2, be2, codebooks, Wd0, bd0, Wd1, bd1, Wd2, bd2):
    B, IN = x.shape
    bs = BLOCK_B
    grid = (B // bs,)
    f32 = jnp.float32
    bf16 = jnp.bfloat16
    cbt = (-2.0 * codebooks).transpose(0, 2, 1)
    cbn = jnp.sum(codebooks * codebooks, axis=2)[:, None, :]
    cb_hi = codebooks.astype(bf16)
    r1 = codebooks - cb_hi.astype(f32)
    cb_mid = r1.astype(bf16)
    cb_lo = (r1 - cb_mid.astype(f32)).astype(bf16)
    OUT = Wd2.shape[1]

    def full(a):
        return pl.BlockSpec(a.shape, lambda i: (0,) * a.ndim)

    b2 = [b.reshape(1, -1) for b in (be0, be1, be2, bd0, bd1, bd2)]

    out_shapes = (
        jax.ShapeDtypeStruct((B, OUT), jnp.float32),
        jax.ShapeDtypeStruct((LEVELS, B, LATENT), jnp.float32),
        jax.ShapeDtypeStruct((LEVELS, B, LATENT), jnp.float32),
        jax.ShapeDtypeStruct((LEVELS, K), jnp.int32),
        jax.ShapeDtypeStruct((B, LEVELS), jnp.int32),
    )
    out_specs = (
        pl.BlockSpec((bs, OUT), lambda i: (i, 0)),
        pl.BlockSpec((LEVELS, bs, LATENT), lambda i: (0, i, 0)),
        pl.BlockSpec((LEVELS, bs, LATENT), lambda i: (0, i, 0)),
        pl.BlockSpec((LEVELS, K), lambda i: (0, 0)),
        pl.BlockSpec((bs, LEVELS), lambda i: (i, 0)),
    )
    in_specs = [
        pl.BlockSpec((bs, IN), lambda i: (i, 0)),
        full(We0), full(b2[0]), full(We1), full(b2[1]), full(We2), full(b2[2]),
        full(cb_hi), full(cb_mid), full(cb_lo), full(cbt), full(cbn),
        full(Wd0), full(b2[3]), full(Wd1), full(b2[4]), full(Wd2), full(b2[5]),
    ]

    decoded, r, e, counts, quantized = pl.pallas_call(
        _rqvae_block,
        grid=grid,
        in_specs=in_specs,
        out_specs=out_specs,
        out_shape=out_shapes,
    )(x, We0, b2[0], We1, b2[1], We2, b2[2], cb_hi, cb_mid, cb_lo, cbt, cbn,
      Wd0, b2[3], Wd1, b2[4], Wd2, b2[5])
    return (decoded, r, e, counts, quantized)


# speedup vs baseline: 2.0694x; 1.4324x over previous
import functools

import jax
import jax.numpy as jnp
from jax.experimental import pallas as pl

LEVELS = 4
K = 1024
LATENT = 64
BLOCK_B = 512


def _rqvae_block(x_ref, We0, be0, We1, be1, We2, be2,
                 cbs_ref, cbt_ref, cbn_ref,
                 Wd0, bd0, Wd1, bd1, Wd2, bd2,
                 dec_ref, r_ref, e_ref, cnt_ref, q_ref):
    f32 = jnp.float32
    bf16 = jnp.bfloat16

    def dot16(a, b):
        return jnp.dot(a.astype(bf16), b.astype(bf16), preferred_element_type=f32)

    x = x_ref[...]
    h = jnp.maximum(dot16(x, We0[...]) + be0[...], 0.0)
    h = jnp.maximum(dot16(h, We1[...]) + be1[...], 0.0)
    z = dot16(h, We2[...]) + be2[...]

    bs = z.shape[0]
    iota_k = jax.lax.broadcasted_iota(jnp.int32, (bs, K), 1)
    ones_row = jnp.ones((1, bs), dtype=bf16)

    residual = z
    z_hat = jnp.zeros_like(z)
    idx_cols, cnt_rows = [], []
    for l in range(LEVELS):
        rnorm = jnp.sum(residual * residual, axis=1, keepdims=True)
        d = (rnorm + dot16(residual, cbt_ref[l])) + cbn_ref[l]
        m = jnp.min(d, axis=1, keepdims=True)
        idx = jnp.min(jnp.where(d == m, iota_k, K), axis=1, keepdims=True)
        onehot = (iota_k == idx).astype(bf16)
        parts = jnp.dot(onehot, cbs_ref[l], preferred_element_type=f32)
        e_l = (parts[:, :LATENT] + parts[:, LATENT:2 * LATENT]) \
              + parts[:, 2 * LATENT:]
        cnt_rows.append(jnp.dot(ones_row, onehot, preferred_element_type=f32))
        idx_cols.append(idx)
        r_ref[l] = residual
        e_ref[l] = e_l
        z_hat = z_hat + e_l
        residual = residual - e_l

    q_ref[...] = jnp.concatenate(idx_cols, axis=1)
    stacked = jnp.concatenate(cnt_rows, axis=0).astype(jnp.int32)

    @pl.when(pl.program_id(0) == 0)
    def _():
        cnt_ref[...] = stacked

    @pl.when(pl.program_id(0) != 0)
    def _():
        cnt_ref[...] = cnt_ref[...] + stacked

    zst = z + (z_hat - z)
    g = jnp.maximum(dot16(zst, Wd0[...]) + bd0[...], 0.0)
    g = jnp.maximum(dot16(g, Wd1[...]) + bd1[...], 0.0)
    dec_ref[...] = dot16(g, Wd2[...]) + bd2[...]


@functools.partial(jax.jit, static_argnames=())
def kernel(x, We0, be0, We1, be1, We2, be2, codebooks, Wd0, bd0, Wd1, bd1, Wd2, bd2):
    B, IN = x.shape
    bs = BLOCK_B
    grid = (B // bs,)
    f32 = jnp.float32
    bf16 = jnp.bfloat16
    cbt = (-2.0 * codebooks).transpose(0, 2, 1)
    cbn = jnp.sum(codebooks * codebooks, axis=2)[:, None, :]
    cb_hi = codebooks.astype(bf16)
    r1 = codebooks - cb_hi.astype(f32)
    cb_mid = r1.astype(bf16)
    cb_lo = (r1 - cb_mid.astype(f32)).astype(bf16)
    cb_split = jnp.concatenate([cb_hi, cb_mid, cb_lo], axis=2)
    OUT = Wd2.shape[1]

    def full(a):
        return pl.BlockSpec(a.shape, lambda i: (0,) * a.ndim)

    b2 = [b.reshape(1, -1) for b in (be0, be1, be2, bd0, bd1, bd2)]

    out_shapes = (
        jax.ShapeDtypeStruct((B, OUT), jnp.float32),
        jax.ShapeDtypeStruct((LEVELS, B, LATENT), jnp.float32),
        jax.ShapeDtypeStruct((LEVELS, B, LATENT), jnp.float32),
        jax.ShapeDtypeStruct((LEVELS, K), jnp.int32),
        jax.ShapeDtypeStruct((B, LEVELS), jnp.int32),
    )
    out_specs = (
        pl.BlockSpec((bs, OUT), lambda i: (i, 0)),
        pl.BlockSpec((LEVELS, bs, LATENT), lambda i: (0, i, 0)),
        pl.BlockSpec((LEVELS, bs, LATENT), lambda i: (0, i, 0)),
        pl.BlockSpec((LEVELS, K), lambda i: (0, 0)),
        pl.BlockSpec((bs, LEVELS), lambda i: (i, 0)),
    )
    in_specs = [
        pl.BlockSpec((bs, IN), lambda i: (i, 0)),
        full(We0), full(b2[0]), full(We1), full(b2[1]), full(We2), full(b2[2]),
        full(cb_split), full(cbt), full(cbn),
        full(Wd0), full(b2[3]), full(Wd1), full(b2[4]), full(Wd2), full(b2[5]),
    ]

    decoded, r, e, counts, quantized = pl.pallas_call(
        _rqvae_block,
        grid=grid,
        in_specs=in_specs,
        out_specs=out_specs,
        out_shape=out_shapes,
    )(x, We0, b2[0], We1, b2[1], We2, b2[2], cb_split, cbt, cbn,
      Wd0, b2[3], Wd1, b2[4], Wd2, b2[5])
    return (decoded, r, e, counts, quantized)
